# Initial kernel scaffold; baseline (speedup 1.0000x reference)
import functools

import jax
import jax.numpy as jnp
from jax import lax
from jax.experimental import pallas as pl
from jax.experimental.pallas import tpu as pltpu
from jax.experimental.pallas import tpu_sc as plsc

D_MODEL = 64
SEQ_LEN = 200
BATCH = 1024
NTOK = BATCH * SEQ_LEN

_info = plsc.get_sparse_core_info()
NC, NS = _info.num_cores, _info.num_subcores
NW = NC * NS
ROWS_PER_W = NTOK // NW
CHUNK = 400
NCHUNK = ROWS_PER_W // CHUNK
SUB = 80
NSUB = CHUNK // SUB
SEQ_PER_CHUNK = CHUNK // SEQ_LEN
VLANES = 16
NCOL = D_MODEL // VLANES


@functools.partial(
    pl.kernel,
    out_type=jax.ShapeDtypeStruct((NTOK, D_MODEL), jnp.float32),
    mesh=plsc.VectorSubcoreMesh(core_axis_name="c", subcore_axis_name="s"),
    scratch_types=[
        pltpu.VMEM((SEQ_LEN, D_MODEL), jnp.float32),
        pltpu.VMEM((2, NSUB, SUB), jnp.int32),
        pltpu.VMEM((2, CHUNK, D_MODEL), jnp.float32),
        pltpu.SemaphoreType.DMA,
        pltpu.SemaphoreType.DMA,
    ],
)
def _emb_lookup(idx_hbm, table_hbm, pos_hbm, out_hbm, pos_v, idx_v, rows_v,
                sem_a, sem_b):
    sems = (sem_a, sem_b)
    wid = lax.axis_index("s") * NC + lax.axis_index("c")

    pltpu.sync_copy(pos_hbm, pos_v)

    def fire(k, b):
        r0 = wid * (ROWS_PER_W // SUB) + k * NSUB
        pltpu.sync_copy(idx_hbm.at[pl.ds(r0, NSUB)], idx_v.at[b])
        for j in range(NSUB):
            pltpu.async_copy(
                table_hbm.at[idx_v.at[b].at[j]],
                rows_v.at[b].at[pl.ds(j * SUB, SUB)],
                sems[b],
            )

    def drain(b):
        pltpu.make_async_copy(
            table_hbm.at[pl.ds(0, CHUNK)], rows_v.at[b], sems[b]
        ).wait()

    def add_pos(b):
        rb = rows_v.at[b]
        for seq in range(SEQ_PER_CHUNK):
            def body(r, carry):
                for c in range(NCOL):
                    sl = pl.ds(c * VLANES, VLANES)
                    rb[seq * SEQ_LEN + r, sl] += pos_v[r, sl]
                return carry
            lax.fori_loop(0, SEQ_LEN, body, 0)

    def store(k, b):
        base = wid * ROWS_PER_W + k * CHUNK
        pltpu.sync_copy(rows_v.at[b], out_hbm.at[pl.ds(base, CHUNK)])

    fire(0, 0)
    for k in range(NCHUNK):
        b = k & 1
        if k + 1 < NCHUNK:
            fire(k + 1, 1 - b)
        drain(b)
        add_pos(b)
        store(k, b)


def kernel(inputs, input_emb_table, positional_emb_table):
    idx = inputs.astype(jnp.int32).reshape(NTOK // SUB, SUB)
    out = _emb_lookup(idx, input_emb_table, positional_emb_table)
    return out.reshape(BATCH, SEQ_LEN, D_MODEL)


# speedup vs baseline: 1.3238x; 1.3238x over previous
import functools

import jax
import jax.numpy as jnp
from jax import lax
from jax.experimental import pallas as pl
from jax.experimental.pallas import tpu as pltpu
from jax.experimental.pallas import tpu_sc as plsc

D_MODEL = 64
SEQ_LEN = 200
BATCH = 1024
NTOK = BATCH * SEQ_LEN

_info = plsc.get_sparse_core_info()
NC, NS = _info.num_cores, _info.num_subcores
NW = NC * NS
ROWS_PER_W = NTOK // NW
CHUNK = 320
NCHUNK = ROWS_PER_W // CHUNK
SUB = 40
NSUB = CHUNK // SUB
PEXT = 480
VLANES = 16
NCOL = D_MODEL // VLANES


@functools.partial(
    pl.kernel,
    out_type=jax.ShapeDtypeStruct((NTOK, D_MODEL), jnp.float32),
    mesh=plsc.VectorSubcoreMesh(core_axis_name="c", subcore_axis_name="s"),
    scratch_types=[
        pltpu.VMEM((PEXT, D_MODEL), jnp.float32),
        pltpu.VMEM((2, NSUB, SUB), jnp.int32),
        pltpu.VMEM((2, CHUNK, D_MODEL), jnp.float32),
        pltpu.SemaphoreType.DMA,
        pltpu.SemaphoreType.DMA,
    ],
    compiler_params=pltpu.CompilerParams(use_tc_tiling_on_sc=False),
)
def _emb_lookup(idx_hbm, table_hbm, pos_hbm, out_hbm, pos_v, idx_v, rows_v,
                sem_a, sem_b):
    sems = (sem_a, sem_b)
    wid = lax.axis_index("s") * NC + lax.axis_index("c")

    pltpu.sync_copy(pos_hbm, pos_v.at[pl.ds(0, SEQ_LEN)])
    pltpu.sync_copy(pos_hbm, pos_v.at[pl.ds(SEQ_LEN, SEQ_LEN)])
    pltpu.sync_copy(pos_hbm.at[pl.ds(0, PEXT - 2 * SEQ_LEN)],
                    pos_v.at[pl.ds(2 * SEQ_LEN, PEXT - 2 * SEQ_LEN)])

    def fire(k, b):
        r0 = wid * (ROWS_PER_W // SUB) + k * NSUB
        pltpu.sync_copy(idx_hbm.at[pl.ds(r0, NSUB)], idx_v.at[b])
        for j in range(NSUB):
            pltpu.async_copy(
                table_hbm.at[idx_v.at[b].at[j]],
                rows_v.at[b].at[pl.ds(j * SUB, SUB)],
                sems[b],
            )

    def drain(b):
        pltpu.make_async_copy(
            table_hbm.at[pl.ds(0, CHUNK)], rows_v.at[b], sems[b]
        ).wait()

    def add_pos(k, b):
        rb = rows_v.at[b]
        p0 = (k * CHUNK) % SEQ_LEN

        def body(r, carry):
            for c in range(NCOL):
                sl = pl.ds(c * VLANES, VLANES)
                rb[r, sl] += pos_v[p0 + r, sl]
            return carry

        lax.fori_loop(0, CHUNK, body, 0)

    def store(k, b):
        base = wid * ROWS_PER_W + k * CHUNK
        pltpu.sync_copy(rows_v.at[b], out_hbm.at[pl.ds(base, CHUNK)])

    fire(0, 0)
    for k in range(NCHUNK):
        b = k & 1
        if k + 1 < NCHUNK:
            fire(k + 1, 1 - b)
        drain(b)
        add_pos(k, b)
        store(k, b)


def kernel(inputs, input_emb_table, positional_emb_table):
    idx = inputs.astype(jnp.int32).reshape(NTOK // SUB, SUB)
    out = _emb_lookup(idx, input_emb_table, positional_emb_table)
    return out.reshape(BATCH, SEQ_LEN, D_MODEL)
